# Initial kernel scaffold; baseline (speedup 1.0000x reference)
import jax
import jax.numpy as jnp
from jax.experimental import pallas as pl


def kernel(x, batch, ws1, bs1, wh1, bh1, wa1, wb1, bb1, ws2, bs2, wh2, bh2, wa2, wb2, bb2, ws3, bs3, wh3, bh3, wa3, wb3, bb3):
    raise NotImplementedError("write your pallas kernel here")


# speedup vs baseline: 4.1475x; 4.1475x over previous
import functools

import jax
import jax.numpy as jnp
from jax.experimental import pallas as pl

K = 20
NG = 16


def _dot3(a, b):
    ah = a.astype(jnp.bfloat16)
    al = (a - ah.astype(jnp.float32)).astype(jnp.bfloat16)
    bh = b.astype(jnp.bfloat16)
    bl = (b - bh.astype(jnp.float32)).astype(jnp.bfloat16)

    def f(u, v):
        return jax.lax.dot_general(u, v, (((1,), (0,)), ((), ())),
                                   preferred_element_type=jnp.float32)

    return f(ah, bh) + (f(ah, bl) + (f(al, bh) + f(al, bl)))


def _dot1(a, b):
    return jax.lax.dot_general(a, b, (((1,), (0,)), ((), ())),
                               preferred_element_type=jnp.float32)


def _row_block(n, cap=256):
    rb = 8
    for c in range(8, cap + 1, 8):
        if n % c == 0:
            rb = c
    return rb


def _proj_body(x_ref, w_ref, b_ref, shp_ref):
    sh = _dot1(x_ref[...], w_ref[...]) + b_ref[...]
    n = sh.shape[0]
    npad = shp_ref.shape[0]
    shp_ref[...] = jnp.concatenate(
        [sh, jnp.full((npad - n, 8), 1e18, jnp.float32)], axis=0)


def _proj(x, wsh, bsh, npad):
    return pl.pallas_call(
        _proj_body,
        out_shape=jax.ShapeDtypeStruct((npad, 8), jnp.float32),
    )(x, wsh, bsh)


def _dist_out_body(shb_ref, sht_ref, shp_ref, x_ref, wa_ref, wb_ref, bb_ref,
                   y_ref, *, leaky):
    sb = shb_ref[...]
    st = sht_ref[...]
    rb = sb.shape[0]
    n = st.shape[1]

    acc = jnp.zeros((rb, n), jnp.float32)
    for c in range(3):
        diff = sb[:, c:c + 1] - st[c:c + 1, :]
        acc = acc + diff * diff

    lane = jax.lax.broadcasted_iota(jnp.int32, (rb, 8), 1)
    sbs = jnp.where(lane < 3, sb, 0.0)
    sub = jax.lax.broadcasted_iota(jnp.int32, (8, n), 0)
    st3 = jnp.where(sub < 3, st, 0.0)
    cross = _dot1(sbs, st3)
    sqb = sb[:, 5:6]
    sqa = st[5:6, :]
    d2s = (sqb + sqa) - 2.0 * cross

    w = jnp.exp(-10.0 * acc)
    wh0 = w * st[3:4, :]
    wh1 = w * st[4:5, :]
    zero1 = jnp.zeros((rb, 1), jnp.float32)

    def body(_, carry):
        wrk, s0, s1 = carry
        m = jnp.min(wrk, axis=1, keepdims=True)
        hit = wrk == m
        s0 = s0 + jnp.sum(jnp.where(hit, wh0, 0.0), axis=1, keepdims=True)
        s1 = s1 + jnp.sum(jnp.where(hit, wh1, 0.0), axis=1, keepdims=True)
        return jnp.where(hit, jnp.float32(jnp.inf), wrk), s0, s1

    wrk, s0, s1 = jax.lax.fori_loop(0, K, body, (d2s, zero1, zero1))
    sel = jnp.isinf(wrk)
    kf = jnp.float32(K)
    neg = jnp.float32(-jnp.inf)
    m0 = jnp.max(jnp.where(sel, wh0, neg), axis=1, keepdims=True)
    m1 = jnp.max(jnp.where(sel, wh1, neg), axis=1, keepdims=True)
    z = jnp.zeros((rb, 4), jnp.float32)
    agg = jnp.concatenate([s0 / kf, s1 / kf, m0, m1, z], axis=1)
    y = _dot1(x_ref[...], wa_ref[...]) + _dot1(agg, wb_ref[...]) + bb_ref[...]
    if leaky:
        y = jnp.where(y >= 0, y, 0.01 * y)
    y_ref[...] = y


def _dist_out(shb, sht, shp, x, wa, wb8, bb, leaky):
    nb, npad = shb.shape[0], sht.shape[1]
    ci = x.shape[1]
    co = wa.shape[1]
    rb = _row_block(nb)
    return pl.pallas_call(
        functools.partial(_dist_out_body, leaky=leaky),
        grid=(nb // rb,),
        in_specs=[
            pl.BlockSpec((rb, 8), lambda i: (i, 0)),
            pl.BlockSpec((8, npad), lambda i: (0, 0)),
            pl.BlockSpec((npad, 8), lambda i: (0, 0)),
            pl.BlockSpec((rb, ci), lambda i: (i, 0)),
            pl.BlockSpec((ci, co), lambda i: (0, 0)),
            pl.BlockSpec((8, co), lambda i: (0, 0)),
            pl.BlockSpec((1, co), lambda i: (0, 0)),
        ],
        out_specs=pl.BlockSpec((rb, co), lambda i: (i, 0)),
        out_shape=jax.ShapeDtypeStruct((nb, co), jnp.float32),
    )(shb, sht, shp, x, wa, wb8, bb)


def _pack_w(ws, wh):
    ci = ws.shape[0]
    wsh = jnp.zeros((ci, 8), jnp.float32)
    wsh = wsh.at[:, 0:3].set(ws).at[:, 3:5].set(wh)
    return wsh


def _pack_b(bs, bh):
    b = jnp.zeros((1, 8), jnp.float32)
    return b.at[0, 0:3].set(bs).at[0, 3:5].set(bh)


def kernel(x, batch, ws1, bs1, wh1, bh1, wa1, wb1, bb1, ws2, bs2, wh2, bh2,
           wa2, wb2, bb2, ws3, bs3, wh3, bh3, wa3, wb3, bb3):
    b = jnp.concatenate([jnp.zeros((1,), batch.dtype), batch])
    d = b[1:] - b[:-1]
    d = d.at[0].set(1)
    idx0 = jnp.nonzero(d, size=NG)[0]

    h = x
    for li, (ws, bs, wh, bh, wa, wb, bb) in enumerate([
            (ws1, bs1, wh1, bh1, wa1, wb1, bb1),
            (ws2, bs2, wh2, bh2, wa2, wb2, bb2),
            (ws3, bs3, wh3, bh3, wa3, wb3, bb3)]):
        n = h.shape[0]
        npad = -(-n // 512) * 512
        shp = _proj(h, _pack_w(ws, wh), _pack_b(bs, bh), npad)
        sq = jnp.sum(shp[:, 0:3] * shp[:, 0:3], axis=1)
        shp = shp.at[:, 5].set(sq)
        sht = shp.T
        wb8 = jnp.concatenate([wb, jnp.zeros((4, wb.shape[1]), jnp.float32)],
                              axis=0)
        if li < 2:
            h = _dist_out(shp[:n], sht, shp, h, wa, wb8, bb[None, :],
                          leaky=True)
        else:
            h = _dist_out(shp[idx0], sht, shp, h[idx0], wa, wb8, bb[None, :],
                          leaky=False)
    return h
